# Initial kernel scaffold; baseline (speedup 1.0000x reference)
import jax
import jax.numpy as jnp
from jax.experimental import pallas as pl


def kernel(coords_a, batch_idx_a, feats_a, coords_b, batch_idx_b, feats_b):
    raise NotImplementedError("write your pallas kernel here")


# speedup vs baseline: 25.1165x; 25.1165x over previous
import functools

import jax
import jax.numpy as jnp
from jax import lax
from jax.experimental import pallas as pl
from jax.experimental.pallas import tpu as pltpu
from jax.experimental.pallas import tpu_sc as plsc

FULL_SCALE = 32
TOPK = 8
R = 0.5
INT_MAX = 2**31 - 1

QT = 128
CC = 2048


def _topk_body(ca_ref, ba_ref, cbt_ref, bb_ref, idx_ref, w_ref):
    nb = cbt_ref.shape[1]
    shift = (nb - 1).bit_length()
    ca = ca_ref[...]
    ba = ba_ref[...]
    an = jnp.sum(ca * ca, axis=1, keepdims=True)
    cols0 = lax.broadcasted_iota(jnp.int32, (QT, CC), 1)

    def chunk_body(c, top8):
        col0 = c * CC
        cb = cbt_ref[:, pl.ds(col0, CC)]
        bb = bb_ref[:, pl.ds(col0, CC)]
        ab = lax.dot_general(ca, cb, (((1,), (0,)), ((), ())),
                             preferred_element_type=jnp.float32,
                             precision=lax.Precision.HIGHEST)
        bn = jnp.sum(cb * cb, axis=0, keepdims=True)
        d2i = (an + bn - 2.0 * ab).astype(jnp.int32)
        key = (d2i << shift) | (cols0 + col0)
        key = jnp.where(ba == bb, key, INT_MAX)
        cand = []
        for _ in range(TOPK):
            m = jnp.min(key, axis=1, keepdims=True)
            cand.append(m)
            key = jnp.where(key == m, INT_MAX, key)
        merged = jnp.concatenate([top8] + cand, axis=1)
        new = []
        for _ in range(TOPK):
            m = jnp.min(merged, axis=1, keepdims=True)
            new.append(m)
            merged = jnp.where(merged == m, INT_MAX, merged)
        return jnp.concatenate(new, axis=1)

    init = jnp.full((QT, TOPK), INT_MAX, jnp.int32)
    top8 = lax.fori_loop(0, nb // CC, chunk_body, init)
    idx_ref[...] = top8 & (nb - 1)
    d2f = (top8 >> shift).astype(jnp.float32)
    w_ref[...] = jnp.maximum(R - jnp.sqrt(d2f) / FULL_SCALE, 0.0)


def _topk_call(ca, ba, cbt, bb):
    na = ca.shape[0]
    nb = cbt.shape[1]
    return pl.pallas_call(
        _topk_body,
        grid=(na // QT,),
        in_specs=[
            pl.BlockSpec((QT, 8), lambda i: (i, 0)),
            pl.BlockSpec((QT, 1), lambda i: (i, 0)),
            pl.BlockSpec((8, nb), lambda i: (0, 0)),
            pl.BlockSpec((1, nb), lambda i: (0, 0)),
        ],
        out_specs=[
            pl.BlockSpec((QT, TOPK), lambda i: (i, 0)),
            pl.BlockSpec((QT, TOPK), lambda i: (i, 0)),
        ],
        out_shape=[
            jax.ShapeDtypeStruct((na, TOPK), jnp.int32),
            jax.ShapeDtypeStruct((na, TOPK), jnp.float32),
        ],
    )(ca, ba, cbt, bb)


def _make_gather_kernel(na, d, nw):
    q_per_w = na // nw
    n_idx = q_per_w * TOPK
    n_rows = n_idx // 128
    sup = 4
    rows_per_sup = n_rows // sup
    q_per_sup = q_per_w // sup
    mesh = plsc.VectorSubcoreMesh(core_axis_name="c", subcore_axis_name="s")

    @functools.partial(
        pl.kernel, mesh=mesh,
        compiler_params=pltpu.CompilerParams(use_tc_tiling_on_sc=False),
        out_type=jax.ShapeDtypeStruct((na, d), jnp.float32),
        scratch_types=[
            pltpu.VMEM((n_rows, 128), jnp.int32),
            pltpu.VMEM((n_idx,), jnp.float32),
            pltpu.VMEM((rows_per_sup * 128, d), jnp.float32),
            pltpu.VMEM((q_per_sup, d), jnp.float32),
            pltpu.SemaphoreType.DMA,
        ],
    )
    def gather_kernel(idx_hbm, w_hbm, feats_hbm, out_hbm,
                      idx_v, w_v, rows_v, out_v, sem):
        wid = lax.axis_index("s") * 2 + lax.axis_index("c")
        pltpu.sync_copy(idx_hbm.at[wid], idx_v)
        pltpu.sync_copy(w_hbm.at[wid], w_v)
        for s in range(sup):
            handles = []
            for b in range(rows_per_sup):
                handles.append(pltpu.async_copy(
                    feats_hbm.at[idx_v.at[s * rows_per_sup + b]],
                    rows_v.at[pl.ds(b * 128, 128)], sem))
            for h in handles:
                h.wait()

            def pbody(p, _, s=s):
                wv = w_v[pl.ds((s * q_per_sup + 2 * p) * TOPK, 16)]
                for j in range(2):
                    q = 2 * p + j
                    acc0 = jnp.zeros((16,), jnp.float32)
                    acc1 = jnp.zeros((16,), jnp.float32)
                    for k in range(TOPK):
                        wk = wv[j * TOPK + k]
                        r = q * TOPK + k
                        acc0 = acc0 + rows_v[r, pl.ds(0, 16)] * wk
                        acc1 = acc1 + rows_v[r, pl.ds(16, 16)] * wk
                    out_v[q, pl.ds(0, 16)] = acc0
                    out_v[q, pl.ds(16, 16)] = acc1
                return 0

            lax.fori_loop(0, q_per_sup // 2, pbody, 0)
            pltpu.sync_copy(
                out_v, out_hbm.at[pl.ds(wid * q_per_w + s * q_per_sup,
                                        q_per_sup)])

    return gather_kernel


def kernel(coords_a, batch_idx_a, feats_a, coords_b, batch_idx_b, feats_b):
    na = coords_a.shape[0]
    nb = coords_b.shape[0]
    d = feats_b.shape[1]
    nw = 32

    ca = jnp.pad(coords_a.astype(jnp.float32), ((0, 0), (0, 5)))
    cbt = jnp.pad(coords_b.astype(jnp.float32), ((0, 0), (0, 5))).T
    ba = batch_idx_a.reshape(na, 1)
    bb = batch_idx_b.reshape(1, nb)

    idx, w = _topk_call(ca, ba, cbt, bb)

    idx3 = idx.reshape(nw, (na // nw) * TOPK // 128, 128)
    w2 = w.reshape(nw, (na // nw) * TOPK)
    tmp = _make_gather_kernel(na, d, nw)(idx3, w2, feats_b)
    return jnp.concatenate([feats_a, tmp], axis=1)


# speedup vs baseline: 49.0551x; 1.9531x over previous
import functools

import jax
import jax.numpy as jnp
from jax import lax
from jax.experimental import pallas as pl
from jax.experimental.pallas import tpu as pltpu
from jax.experimental.pallas import tpu_sc as plsc

FULL_SCALE = 32
TOPK = 8
R = 0.5
INT_MAX = 2**31 - 1

QT = 128
CC = 2048


def _topk_body(ca_ref, ba_ref, cbt_ref, bb_ref, idx_ref, w_ref):
    nb = cbt_ref.shape[1]
    shift = (nb - 1).bit_length()
    ca = ca_ref[...]
    ba = ba_ref[...]
    an = jnp.sum(ca * ca, axis=1, keepdims=True)
    cols0 = lax.broadcasted_iota(jnp.int32, (QT, CC), 1)

    def chunk_body(c, top8):
        col0 = c * CC
        cb = cbt_ref[:, pl.ds(col0, CC)]
        bb = bb_ref[:, pl.ds(col0, CC)]
        ab = lax.dot_general(ca, cb, (((1,), (0,)), ((), ())),
                             preferred_element_type=jnp.float32,
                             precision=lax.Precision.HIGHEST)
        bn = jnp.sum(cb * cb, axis=0, keepdims=True)
        d2i = (an + bn - 2.0 * ab).astype(jnp.int32)
        key = (d2i << shift) | (cols0 + col0)
        key = jnp.where(ba == bb, key, INT_MAX)
        cand = []
        for _ in range(TOPK):
            m = jnp.min(key, axis=1, keepdims=True)
            cand.append(m)
            key = jnp.where(key == m, INT_MAX, key)
        merged = jnp.concatenate([top8] + cand, axis=1)
        new = []
        for _ in range(TOPK):
            m = jnp.min(merged, axis=1, keepdims=True)
            new.append(m)
            merged = jnp.where(merged == m, INT_MAX, merged)
        return jnp.concatenate(new, axis=1)

    qmin = ba[0, 0]
    qmax = ba[QT - 1, 0]
    bbrow = bb_ref[...]
    lo = jnp.sum((bbrow < qmin).astype(jnp.int32))
    hi = jnp.sum((bbrow <= qmax).astype(jnp.int32))
    clo = lo // CC
    chi = (hi + CC - 1) // CC
    init = jnp.full((QT, TOPK), INT_MAX, jnp.int32)
    top8 = lax.fori_loop(clo, chi, chunk_body, init)
    idx_ref[...] = top8 & (nb - 1)
    d2f = (top8 >> shift).astype(jnp.float32)
    w_ref[...] = jnp.maximum(R - jnp.sqrt(d2f) / FULL_SCALE, 0.0)


def _topk_call(ca, ba, cbt, bb):
    na = ca.shape[0]
    nb = cbt.shape[1]
    return pl.pallas_call(
        _topk_body,
        grid=(na // QT,),
        in_specs=[
            pl.BlockSpec((QT, 8), lambda i: (i, 0)),
            pl.BlockSpec((QT, 1), lambda i: (i, 0)),
            pl.BlockSpec((8, nb), lambda i: (0, 0)),
            pl.BlockSpec((1, nb), lambda i: (0, 0)),
        ],
        out_specs=[
            pl.BlockSpec((QT, TOPK), lambda i: (i, 0)),
            pl.BlockSpec((QT, TOPK), lambda i: (i, 0)),
        ],
        out_shape=[
            jax.ShapeDtypeStruct((na, TOPK), jnp.int32),
            jax.ShapeDtypeStruct((na, TOPK), jnp.float32),
        ],
    )(ca, ba, cbt, bb)


def _make_gather_kernel(na, d, nw):
    q_per_w = na // nw
    n_idx = q_per_w * TOPK
    n_rows = n_idx // 128
    sup = 4
    rows_per_sup = n_rows // sup
    q_per_sup = q_per_w // sup
    mesh = plsc.VectorSubcoreMesh(core_axis_name="c", subcore_axis_name="s")

    @functools.partial(
        pl.kernel, mesh=mesh,
        compiler_params=pltpu.CompilerParams(use_tc_tiling_on_sc=False),
        out_type=jax.ShapeDtypeStruct((na, d), jnp.float32),
        scratch_types=[
            pltpu.VMEM((n_rows, 128), jnp.int32),
            pltpu.VMEM((n_idx,), jnp.float32),
            pltpu.VMEM((rows_per_sup * 128, d), jnp.float32),
            pltpu.VMEM((q_per_sup, d), jnp.float32),
            pltpu.SemaphoreType.DMA,
        ],
    )
    def gather_kernel(idx_hbm, w_hbm, feats_hbm, out_hbm,
                      idx_v, w_v, rows_v, out_v, sem):
        wid = lax.axis_index("s") * 2 + lax.axis_index("c")
        pltpu.sync_copy(idx_hbm.at[wid], idx_v)
        pltpu.sync_copy(w_hbm.at[wid], w_v)
        for s in range(sup):
            handles = []
            for b in range(rows_per_sup):
                handles.append(pltpu.async_copy(
                    feats_hbm.at[idx_v.at[s * rows_per_sup + b]],
                    rows_v.at[pl.ds(b * 128, 128)], sem))
            for h in handles:
                h.wait()

            def pbody(p, _, s=s):
                wv = w_v[pl.ds((s * q_per_sup + 2 * p) * TOPK, 16)]
                for j in range(2):
                    q = 2 * p + j
                    acc0 = jnp.zeros((16,), jnp.float32)
                    acc1 = jnp.zeros((16,), jnp.float32)
                    for k in range(TOPK):
                        wk = wv[j * TOPK + k]
                        r = q * TOPK + k
                        acc0 = acc0 + rows_v[r, pl.ds(0, 16)] * wk
                        acc1 = acc1 + rows_v[r, pl.ds(16, 16)] * wk
                    out_v[q, pl.ds(0, 16)] = acc0
                    out_v[q, pl.ds(16, 16)] = acc1
                return 0

            lax.fori_loop(0, q_per_sup // 2, pbody, 0)
            pltpu.sync_copy(
                out_v, out_hbm.at[pl.ds(wid * q_per_w + s * q_per_sup,
                                        q_per_sup)])

    return gather_kernel


def kernel(coords_a, batch_idx_a, feats_a, coords_b, batch_idx_b, feats_b):
    na = coords_a.shape[0]
    nb = coords_b.shape[0]
    d = feats_b.shape[1]
    nw = 32

    ca = jnp.pad(coords_a.astype(jnp.float32), ((0, 0), (0, 5)))
    cbt = jnp.pad(coords_b.astype(jnp.float32), ((0, 0), (0, 5))).T
    ba = batch_idx_a.reshape(na, 1)
    bb = batch_idx_b.reshape(1, nb)

    idx, w = _topk_call(ca, ba, cbt, bb)

    idx3 = idx.reshape(nw, (na // nw) * TOPK // 128, 128)
    w2 = w.reshape(nw, (na // nw) * TOPK)
    tmp = _make_gather_kernel(na, d, nw)(idx3, w2, feats_b)
    return jnp.concatenate([feats_a, tmp], axis=1)
